# baseline (speedup 1.0000x reference)
import functools

import jax
import jax.numpy as jnp
from jax import lax
from jax.experimental import pallas as pl
from jax.experimental.pallas import tpu as pltpu
from jax.experimental.pallas import tpu_sc as plsc

NC = 2
NS = 16
NW = NC * NS
L = 16
D = 128


@functools.partial(jax.jit, static_argnames=("E", "C"))
def _sc_edge_dot(embedding_1, embedding_2, packed_idx, *, E, C):
    epw = E // NW
    T = epw // C
    assert T % 2 == 1 and T >= 3
    n_nodes = embedding_1.shape[0]
    npw = n_nodes // NS

    mesh = plsc.VectorSubcoreMesh(core_axis_name="c", subcore_axis_name="s")

    @functools.partial(
        pl.kernel,
        out_type=jax.ShapeDtypeStruct((E,), jnp.float32),
        mesh=mesh,
        scratch_types=[
            pltpu.VMEM((epw,), jnp.int32),
            pltpu.VMEM((C,), jnp.int32),
            pltpu.VMEM((C,), jnp.int32),
            pltpu.VMEM((C,), jnp.int32),
            pltpu.VMEM((C,), jnp.int32),
            pltpu.VMEM((C, D // 2), jnp.int32),
            pltpu.VMEM((C, D // 2), jnp.int32),
            pltpu.VMEM((C, D // 2), jnp.int32),
            pltpu.VMEM((C, D // 2), jnp.int32),
            pltpu.VMEM((epw,), jnp.float32),
            pltpu.VMEM((C * L,), jnp.float32),
            pltpu.VMEM_SHARED((10000, D // 2), jnp.int32),
            pltpu.VMEM_SHARED((10000, D // 2), jnp.int32),
            pltpu.SemaphoreType.DMA,
            pltpu.SemaphoreType.DMA,
            pltpu.SemaphoreType.DMA,
        ],
        compiler_params=pltpu.CompilerParams(needs_layout_passes=False,
                                             use_tc_tiling_on_sc=False),
    )
    def k(e1_hbm, e2_hbm, pidx_hbm, out_hbm,
          idxp_v, idx1a, idx2a, idx1b, idx2b,
          rows1a, rows2a, rows1b, rows2b,
          out_v, stage_v, sh1, sh2, gsema, gsemb, psem):
        sid = lax.axis_index("s")
        wid = sid * NC + lax.axis_index("c")
        w_base = wid * epw

        nb = sid * npw
        cp1 = pltpu.async_copy(e1_hbm.at[pl.ds(nb, npw)],
                               sh1.at[pl.ds(nb, npw)], psem)
        cp2 = pltpu.async_copy(e2_hbm.at[pl.ds(nb, npw)],
                               sh2.at[pl.ds(nb, npw)], psem)
        cp3 = pltpu.async_copy(pidx_hbm.at[pl.ds(w_base, epw)], idxp_v, psem)
        cp1.wait()
        cp2.wait()
        cp3.wait()
        plsc.subcore_barrier()

        def start_gather(t, idx1_v, idx2_v, rows1_v, rows2_v, gsem):
            for w in range(C // L):
                v = idxp_v[pl.ds(t * C + w * L, L)]
                idx1_v[pl.ds(w * L, L)] = v & 0xFFFF
                idx2_v[pl.ds(w * L, L)] = lax.shift_right_logical(v, 16)
            h = C // 2
            for o in (0, h):
                pltpu.async_copy(sh1.at[idx1_v.at[pl.ds(o, h)]],
                                 rows1_v.at[pl.ds(o, h)], gsem)
                pltpu.async_copy(sh2.at[idx2_v.at[pl.ds(o, h)]],
                                 rows2_v.at[pl.ds(o, h)], gsem)

        def wait_gather(idx1_v, idx2_v, rows1_v, rows2_v, gsem):
            h = C // 2
            for o in (0, h):
                pltpu.make_async_copy(sh1.at[idx1_v.at[pl.ds(o, h)]],
                                      rows1_v.at[pl.ds(o, h)], gsem).wait()
                pltpu.make_async_copy(sh2.at[idx2_v.at[pl.ds(o, h)]],
                                      rows2_v.at[pl.ds(o, h)], gsem).wait()

        lanes = lax.iota(jnp.int32, L)
        idxs = [lanes * L + j for j in range(L)]

        def compute_chunk(t, rows1_v, rows2_v):
            ebase = t * C

            @plsc.parallel_loop(0, C, unroll=8)
            def edge_body(e):
                ps = []
                for j in range(D // (2 * L)):
                    v1 = plsc.bitcast(rows1_v[e, pl.ds(j * L, L)],
                                      jnp.bfloat16)
                    v2 = plsc.bitcast(rows2_v[e, pl.ds(j * L, L)],
                                      jnp.bfloat16)
                    ps.append(v1 * v2)
                s = (ps[0] + ps[1]) + (ps[2] + ps[3])
                s_lo, s_hi = plsc.unpack(s, format=plsc.PackFormat.INTERLEAVED)
                stage_v[pl.ds(e * L, L)] = s_lo + s_hi

            @plsc.parallel_loop(0, C // L, unroll=2)
            def group_body(g):
                sub = stage_v.at[pl.ds(g * (L * L), L * L)]
                vs = [plsc.load_gather(sub, [idxs[j]]) for j in range(L)]
                for step in (8, 4, 2, 1):
                    vs = [vs[i] + vs[i + step] for i in range(step)]
                out_v[pl.ds(ebase + g * L, L)] = vs[0]

        start_gather(0, idx1a, idx2a, rows1a, rows2a, gsema)

        def pair_body(p, carry):
            t0 = 2 * p
            start_gather(t0 + 1, idx1b, idx2b, rows1b, rows2b, gsemb)
            wait_gather(idx1a, idx2a, rows1a, rows2a, gsema)
            compute_chunk(t0, rows1a, rows2a)
            start_gather(t0 + 2, idx1a, idx2a, rows1a, rows2a, gsema)
            wait_gather(idx1b, idx2b, rows1b, rows2b, gsemb)
            compute_chunk(t0 + 1, rows1b, rows2b)
            return carry

        lax.fori_loop(0, (T - 1) // 2, pair_body, 0)

        wait_gather(idx1a, idx2a, rows1a, rows2a, gsema)
        compute_chunk(T - 1, rows1a, rows2a)

        pltpu.sync_copy(out_v, out_hbm.at[pl.ds(w_base, epw)])

    return k(embedding_1, embedding_2, packed_idx)


def kernel(embedding_1, embedding_2, edge_label_index):
    E = edge_label_index.shape[1]
    src = edge_label_index[0].astype(jnp.int32)
    dst = edge_label_index[1].astype(jnp.int32)
    packed = src | (dst << 16)
    n = embedding_1.shape[0]
    e1i = jax.lax.bitcast_convert_type(
        embedding_1.astype(jnp.bfloat16).reshape(n, D // 2, 2), jnp.int32)
    e2i = jax.lax.bitcast_convert_type(
        embedding_2.astype(jnp.bfloat16).reshape(n, D // 2, 2), jnp.int32)
    return _sc_edge_dot(e1i, e2i, packed, E=E, C=80)


# speedup vs baseline: 1.0049x; 1.0049x over previous
import functools

import jax
import jax.numpy as jnp
from jax import lax
from jax.experimental import pallas as pl
from jax.experimental.pallas import tpu as pltpu
from jax.experimental.pallas import tpu_sc as plsc

NC = 2
NS = 16
NW = NC * NS
L = 16
D = 128


@functools.partial(jax.jit, static_argnames=("E", "C"))
def _sc_edge_dot(embedding_1, embedding_2, packed_idx, *, E, C):
    epw = E // NW
    T = epw // C
    assert T % 2 == 1 and T >= 3
    n_nodes = embedding_1.shape[0]
    npw = n_nodes // NS

    mesh = plsc.VectorSubcoreMesh(core_axis_name="c", subcore_axis_name="s")

    @functools.partial(
        pl.kernel,
        out_type=jax.ShapeDtypeStruct((E,), jnp.float32),
        mesh=mesh,
        scratch_types=[
            pltpu.VMEM((epw,), jnp.int32),
            pltpu.VMEM((C,), jnp.int32),
            pltpu.VMEM((C,), jnp.int32),
            pltpu.VMEM((C,), jnp.int32),
            pltpu.VMEM((C,), jnp.int32),
            pltpu.VMEM((C, D // 2), jnp.int32),
            pltpu.VMEM((C, D // 2), jnp.int32),
            pltpu.VMEM((C, D // 2), jnp.int32),
            pltpu.VMEM((C, D // 2), jnp.int32),
            pltpu.VMEM((epw,), jnp.float32),
            pltpu.VMEM((C * L,), jnp.float32),
            pltpu.VMEM_SHARED((10000, D // 2), jnp.int32),
            pltpu.VMEM_SHARED((10000, D // 2), jnp.int32),
            pltpu.SemaphoreType.DMA,
            pltpu.SemaphoreType.DMA,
            pltpu.SemaphoreType.DMA,
        ],
        compiler_params=pltpu.CompilerParams(needs_layout_passes=False,
                                             use_tc_tiling_on_sc=False),
    )
    def k(e1_hbm, e2_hbm, pidx_hbm, out_hbm,
          idxp_v, idx1a, idx2a, idx1b, idx2b,
          rows1a, rows2a, rows1b, rows2b,
          out_v, stage_v, sh1, sh2, gsema, gsemb, psem):
        sid = lax.axis_index("s")
        wid = sid * NC + lax.axis_index("c")
        w_base = wid * epw

        nb = sid * npw
        cp1 = pltpu.async_copy(e1_hbm.at[pl.ds(nb, npw)],
                               sh1.at[pl.ds(nb, npw)], psem)
        cp2 = pltpu.async_copy(e2_hbm.at[pl.ds(nb, npw)],
                               sh2.at[pl.ds(nb, npw)], psem)
        cp3 = pltpu.async_copy(pidx_hbm.at[pl.ds(w_base, epw)], idxp_v, psem)
        cp1.wait()
        cp2.wait()
        cp3.wait()
        plsc.subcore_barrier()

        def start_gather(t, idx1_v, idx2_v, rows1_v, rows2_v, gsem):
            for w in range(C // L):
                v = idxp_v[pl.ds(t * C + w * L, L)]
                idx1_v[pl.ds(w * L, L)] = v & 0xFFFF
                idx2_v[pl.ds(w * L, L)] = lax.shift_right_logical(v, 16)
            pltpu.async_copy(sh1.at[idx1_v], rows1_v, gsem)
            pltpu.async_copy(sh2.at[idx2_v], rows2_v, gsem)

        def wait_gather(idx1_v, idx2_v, rows1_v, rows2_v, gsem):
            pltpu.make_async_copy(sh1.at[idx1_v], rows1_v, gsem).wait()
            pltpu.make_async_copy(sh2.at[idx2_v], rows2_v, gsem).wait()

        lanes = lax.iota(jnp.int32, L)
        idxs = [lanes * L + j for j in range(L)]

        def compute_chunk(t, rows1_v, rows2_v):
            ebase = t * C

            @plsc.parallel_loop(0, C, unroll=8)
            def edge_body(e):
                ps = []
                for j in range(D // (2 * L)):
                    v1 = plsc.bitcast(rows1_v[e, pl.ds(j * L, L)],
                                      jnp.bfloat16)
                    v2 = plsc.bitcast(rows2_v[e, pl.ds(j * L, L)],
                                      jnp.bfloat16)
                    ps.append(v1 * v2)
                s = (ps[0] + ps[1]) + (ps[2] + ps[3])
                s_lo, s_hi = plsc.unpack(s, format=plsc.PackFormat.INTERLEAVED)
                stage_v[pl.ds(e * L, L)] = s_lo + s_hi

            @plsc.parallel_loop(0, C // L, unroll=2)
            def group_body(g):
                sub = stage_v.at[pl.ds(g * (L * L), L * L)]
                vs = [plsc.load_gather(sub, [idxs[j]]) for j in range(L)]
                for step in (8, 4, 2, 1):
                    vs = [vs[i] + vs[i + step] for i in range(step)]
                out_v[pl.ds(ebase + g * L, L)] = vs[0]

        start_gather(0, idx1a, idx2a, rows1a, rows2a, gsema)

        def pair_body(p, carry):
            t0 = 2 * p
            start_gather(t0 + 1, idx1b, idx2b, rows1b, rows2b, gsemb)
            wait_gather(idx1a, idx2a, rows1a, rows2a, gsema)
            compute_chunk(t0, rows1a, rows2a)
            start_gather(t0 + 2, idx1a, idx2a, rows1a, rows2a, gsema)
            wait_gather(idx1b, idx2b, rows1b, rows2b, gsemb)
            compute_chunk(t0 + 1, rows1b, rows2b)
            return carry

        lax.fori_loop(0, (T - 1) // 2, pair_body, 0)

        wait_gather(idx1a, idx2a, rows1a, rows2a, gsema)
        compute_chunk(T - 1, rows1a, rows2a)

        pltpu.sync_copy(out_v, out_hbm.at[pl.ds(w_base, epw)])

    return k(embedding_1, embedding_2, packed_idx)


def kernel(embedding_1, embedding_2, edge_label_index):
    E = edge_label_index.shape[1]
    src = edge_label_index[0].astype(jnp.int32)
    dst = edge_label_index[1].astype(jnp.int32)
    packed = src | (dst << 16)
    n = embedding_1.shape[0]
    e1i = jax.lax.bitcast_convert_type(
        embedding_1.astype(jnp.bfloat16).reshape(n, D // 2, 2), jnp.int32)
    e2i = jax.lax.bitcast_convert_type(
        embedding_2.astype(jnp.bfloat16).reshape(n, D // 2, 2), jnp.int32)
    return _sc_edge_dot(e1i, e2i, packed, E=E, C=80)
